# Initial kernel scaffold; baseline (speedup 1.0000x reference)
import jax
import jax.numpy as jnp
from jax.experimental import pallas as pl


def kernel(smiles, sequence, e3fp, ergfp, pubfp, maccsfp, mol_x, mol_edge_index, mol_batch, pro_x, pro_edge_index, pro_batch, smi_m, seq_m, params):
    raise NotImplementedError("write your pallas kernel here")


# speedup vs baseline: 4.8770x; 4.8770x over previous
import functools

import numpy as np

import jax
import jax.numpy as jnp
from jax import lax
from jax.experimental import pallas as pl
from jax.experimental.pallas import tpu as pltpu
from jax.experimental.pallas import tpu_sc as plsc

NC = 2
NS = 16
NW = NC * NS
L = 16

N_NODES = 10000
E_RAW = 160000
E_REAL = E_RAW + N_NODES
K_EDGE = 64
NCH = 88
PIECE = 8
NPIECE = NCH // PIECE
PER_TILE = NCH * K_EDGE
E_PAD = NW * PER_TILE
N_PAD = 10112
ROWS_PT = N_PAD // NS
ZR = 8

_f32 = jnp.float32


def _pad16(d):
  return ((d + 15) // 16) * 16


def _splat(v16, t):
  it16 = lax.broadcasted_iota(jnp.int32, (L,), 0)
  s = jnp.sum(jnp.where(it16 == t, v16, 0.0))
  return jnp.full((L,), s, _f32)


@functools.partial(jax.jit, static_argnames=("pd", "n_halves", "mode"))
def _sc_edges(src, dst, a0, a1, halves, pd, n_halves, mode, wtab=None):
  mesh = plsc.VectorSubcoreMesh(core_axis_name="c", subcore_axis_name="s",
                                num_cores=NC, num_subcores=NS)

  if mode == "logit":
    outs = [jax.ShapeDtypeStruct((NW, NCH, K_EDGE), _f32)]
  else:
    outs = []
    if mode != "deg":
      outs.append(jax.ShapeDtypeStruct((n_halves * NC, N_PAD, pd), _f32))
    if mode != "gcn":
      outs.append(jax.ShapeDtypeStruct((NC, NS, N_NODES), _f32))

  scratch = [
      pltpu.VMEM((PIECE, K_EDGE), jnp.int32),
      pltpu.VMEM((PIECE, K_EDGE), jnp.int32),
      pltpu.VMEM((N_NODES,), _f32),
      pltpu.VMEM((NCH, K_EDGE), _f32),
      pltpu.VMEM((N_NODES,), _f32),
      pltpu.VMEM((K_EDGE, pd), _f32),
      pltpu.VMEM((ZR, pd), _f32),
      pltpu.VMEM_SHARED((N_PAD, pd), _f32),
      pltpu.SemaphoreType.DMA,
  ]

  def body(*refs):
    it = iter(refs)
    src_h = next(it)
    dst_h = next(it)
    a0_h = next(it)
    a1_h = next(it)
    h_hs = [next(it) for _ in range(len(halves))]
    w_hbm = next(it) if mode == "gat" else None
    if mode == "logit":
      e_o = next(it)
      acc_o = s_o = None
    else:
      acc_o = next(it) if mode != "deg" else None
      s_o = next(it) if mode != "gcn" else None
    sidx_v = next(it)
    didx_v = next(it)
    a_v = next(it)
    w_v = next(it)
    s_v = next(it)
    rows_v = next(it)
    zero_v = next(it)
    acc_sp = next(it)
    sem = next(it)

    cid = lax.axis_index("c")
    sid = lax.axis_index("s")
    wid = sid * NC + cid

    iota = lax.broadcasted_iota(jnp.int32, (L,), 0)
    zero16 = iota.astype(_f32) * 0.0

    if mode in ("deg", "gat"):
      def zs(i, c):
        s_v[pl.ds(i * L, L)] = zero16
        return c
      lax.fori_loop(0, N_NODES // L, zs, 0)

    tile_base = wid * PER_TILE

    if mode == "deg":
      def p1o(p, c):
        pltpu.sync_copy(dst_h.at[wid, pl.ds(p * PIECE, PIECE)], didx_v)
        def p1(j, c2):
          for q in range(K_EDGE // L):
            gid = (tile_base + (p * PIECE + j) * K_EDGE + q * L) + iota
            w = jnp.where(gid < E_REAL, 1.0, 0.0).astype(_f32)
            dz = didx_v[j, pl.ds(q * L, L)]
            plsc.addupdate_scatter(s_v, [dz], w)
          return c2
        lax.fori_loop(0, PIECE, p1, 0)
        return c
      lax.fori_loop(0, NPIECE, p1o, 0)
    elif mode == "gcn":
      pltpu.sync_copy(a0_h, a_v)
      def p1o(p, c):
        pltpu.sync_copy(src_h.at[wid, pl.ds(p * PIECE, PIECE)], sidx_v)
        pltpu.sync_copy(dst_h.at[wid, pl.ds(p * PIECE, PIECE)], didx_v)
        def p1(j, c2):
          for q in range(K_EDGE // L):
            gid = (tile_base + (p * PIECE + j) * K_EDGE + q * L) + iota
            sz = sidx_v[j, pl.ds(q * L, L)]
            dz = didx_v[j, pl.ds(q * L, L)]
            w = plsc.load_gather(a_v, [sz]) * plsc.load_gather(a_v, [dz])
            w = jnp.where(gid < E_REAL, w, 0.0)
            w_v[p * PIECE + j, pl.ds(q * L, L)] = w
          return c2
        lax.fori_loop(0, PIECE, p1, 0)
        return c
      lax.fori_loop(0, NPIECE, p1o, 0)
    elif mode == "logit":
      pltpu.sync_copy(a0_h, a_v)
      def pao(p, c):
        pltpu.sync_copy(src_h.at[wid, pl.ds(p * PIECE, PIECE)], sidx_v)
        def pa(j, c2):
          for q in range(K_EDGE // L):
            sz = sidx_v[j, pl.ds(q * L, L)]
            w_v[p * PIECE + j, pl.ds(q * L, L)] = plsc.load_gather(a_v, [sz])
          return c2
        lax.fori_loop(0, PIECE, pa, 0)
        return c
      lax.fori_loop(0, NPIECE, pao, 0)
      pltpu.sync_copy(a1_h, a_v)
      def pbo(p, c):
        pltpu.sync_copy(dst_h.at[wid, pl.ds(p * PIECE, PIECE)], didx_v)
        def pb(j, c2):
          for q in range(K_EDGE // L):
            dz = didx_v[j, pl.ds(q * L, L)]
            e = (w_v[p * PIECE + j, pl.ds(q * L, L)] +
                 plsc.load_gather(a_v, [dz]))
            e = jnp.maximum(e, 0.2 * e)
            w_v[p * PIECE + j, pl.ds(q * L, L)] = e
          return c2
        lax.fori_loop(0, PIECE, pb, 0)
        return c
      lax.fori_loop(0, NPIECE, pbo, 0)
      pltpu.sync_copy(w_v, e_o.at[wid])
      return
    else:
      pltpu.sync_copy(w_hbm.at[wid], w_v)
      def pbo(p, c):
        pltpu.sync_copy(dst_h.at[wid, pl.ds(p * PIECE, PIECE)], didx_v)
        def pb(j, c2):
          for q in range(K_EDGE // L):
            dz = didx_v[j, pl.ds(q * L, L)]
            w = w_v[p * PIECE + j, pl.ds(q * L, L)]
            plsc.addupdate_scatter(s_v, [dz], w)
          return c2
        lax.fori_loop(0, PIECE, pb, 0)
        return c
      lax.fori_loop(0, NPIECE, pbo, 0)

    if mode != "gcn":
      pltpu.sync_copy(s_v, s_o.at[cid].at[sid])

    if mode == "deg":
      return

    def zz(r, c):
      for cc in range(pd // L):
        zero_v[r, pl.ds(cc * L, L)] = zero16
      return c
    lax.fori_loop(0, ZR, zz, 0)

    for h in range(len(halves)):
      def zacc(r, c):
        pltpu.sync_copy(zero_v, acc_sp.at[pl.ds(sid * ROWS_PT + r * ZR, ZR)])
        return c
      lax.fori_loop(0, ROWS_PT // ZR, zacc, 0)
      plsc.subcore_barrier()

      def p2o(p, c, h=h):
        pltpu.sync_copy(src_h.at[wid, pl.ds(p * PIECE, PIECE)], sidx_v)
        pltpu.sync_copy(dst_h.at[wid, pl.ds(p * PIECE, PIECE)], didx_v)

        def pass2(j, c2):
          pltpu.async_copy(h_hs[h].at[sidx_v.at[j]], rows_v, sem).wait()
          for q in range(K_EDGE // L):
            w16 = w_v[p * PIECE + j, pl.ds(q * L, L)]
            for t in range(L):
              ws = _splat(w16, t)
              k = q * L + t
              for cc in range(pd // L):
                rows_v[k, pl.ds(cc * L, L)] = (
                    rows_v[k, pl.ds(cc * L, L)] * ws)
          pltpu.sync_copy(rows_v, acc_sp.at[didx_v.at[j]], add=True)
          return c2
        lax.fori_loop(0, PIECE, pass2, 0)
        return c
      lax.fori_loop(0, NPIECE, p2o, 0)

      plsc.subcore_barrier()
      def cout(r, c, h=h):
        sl = pl.ds(sid * ROWS_PT + r * ZR, ZR)
        pltpu.sync_copy(acc_sp.at[sl], acc_o.at[h * NC + cid].at[sl])
        return c
      lax.fori_loop(0, ROWS_PT // ZR, cout, 0)

  kern = pl.kernel(body, out_type=tuple(outs), mesh=mesh,
                   scratch_types=scratch,
                   compiler_params=pltpu.CompilerParams(
                       needs_layout_passes=False))
  ins = (src, dst, a0, a1) + tuple(halves)
  if mode == "gat":
    ins = ins + (wtab,)
  res = kern(*ins)
  one = res[0] if isinstance(res, (tuple, list)) else res
  if mode == "logit":
    return one
  if mode == "deg":
    return None, one
  if mode == "gcn":
    return one, None
  return res[0], res[1]


def _tc_exp(e):
  rows = E_PAD // 128
  flat = e.reshape(rows, 128)

  def body(e_ref, o_ref):
    fid = (lax.broadcasted_iota(jnp.int32, (rows, 128), 0) * 128 +
           lax.broadcasted_iota(jnp.int32, (rows, 128), 1))
    o_ref[...] = jnp.where(fid < E_REAL, jnp.exp(e_ref[...]), 0.0)

  out = pl.pallas_call(
      body,
      out_shape=jax.ShapeDtypeStruct((rows, 128), _f32),
  )(flat)
  return out.reshape(NW, NCH, K_EDGE)


_RB = 1000


def _tc_matmul(x, w):
  n, a = x.shape
  b = w.shape[1]

  def body(x_ref, w_ref, o_ref):
    o_ref[...] = jnp.dot(x_ref[...], w_ref[...],
                         preferred_element_type=_f32)

  return pl.pallas_call(
      body,
      grid=(n // _RB,),
      in_specs=[pl.BlockSpec((_RB, a), lambda i: (i, 0)),
                pl.BlockSpec((a, b), lambda i: (0, 0))],
      out_specs=pl.BlockSpec((_RB, b), lambda i: (i, 0)),
      out_shape=jax.ShapeDtypeStruct((n, b), _f32),
  )(x, w)


def _tc_sum32(s_part, rsqrt=False):
  def body(s_ref, o_ref):
    v = jnp.sum(s_ref[...], axis=(0, 1))
    if rsqrt:
      v = 1.0 / jnp.sqrt(v)
    o_ref[...] = jnp.broadcast_to(v[:, None], (N_NODES, 8))

  return pl.pallas_call(
      body,
      out_shape=jax.ShapeDtypeStruct((N_NODES, 8), _f32),
  )(s_part)


def _hw(d, h):
  return min(128, d - 128 * h)


def _tc_gcn_finish(acc, b1, x, w2p, a2s, a2d, d1, n_h2):
  n = N_NODES
  d2 = w2p.shape[0]
  p2_tot = w2p.shape[1]

  def body(acc_ref, b1_ref, x_ref, w2_ref, as_ref, ad_ref, *out_refs):
    agg = acc_ref[0] + acc_ref[1]
    x1 = jax.nn.relu(agg[:, :d1] + b1_ref[...])
    out_refs[0][...] = x1
    cat = jnp.concatenate([x1, x_ref[...]], axis=1)
    h2 = jnp.dot(cat, w2_ref[...], preferred_element_type=_f32)
    for hh in range(n_h2):
      out_refs[1 + hh][...] = h2[:, hh * 128:(hh + 1) * 128]
    asrc = jnp.dot(h2, as_ref[...], preferred_element_type=_f32)
    adst = jnp.dot(h2, ad_ref[...], preferred_element_type=_f32)
    out_refs[1 + n_h2][...] = jnp.broadcast_to(asrc, (_RB, 8))
    out_refs[2 + n_h2][...] = jnp.broadcast_to(adst, (_RB, 8))

  out_specs = ([pl.BlockSpec((_RB, d1), lambda i: (i, 0))] +
               [pl.BlockSpec((_RB, 128), lambda i: (i, 0))] * n_h2 +
               [pl.BlockSpec((_RB, 8), lambda i: (i, 0))] * 2)
  out_shape = ([jax.ShapeDtypeStruct((n, d1), _f32)] +
               [jax.ShapeDtypeStruct((n, 128), _f32)] * n_h2 +
               [jax.ShapeDtypeStruct((n, 8), _f32)] * 2)
  return pl.pallas_call(
      body,
      grid=(n // _RB,),
      in_specs=[pl.BlockSpec((2, _RB, 128), lambda i: (0, i, 0)),
                pl.BlockSpec((1, d1), lambda i: (0, 0)),
                pl.BlockSpec((_RB, d1), lambda i: (i, 0)),
                pl.BlockSpec((d2, p2_tot), lambda i: (0, 0)),
                pl.BlockSpec((p2_tot, 1), lambda i: (0, 0)),
                pl.BlockSpec((p2_tot, 1), lambda i: (0, 0))],
      out_specs=out_specs,
      out_shape=out_shape,
  )(acc, b1, x, w2p, a2s, a2d)


def _tc_gat_finish(acc, s_part, b2, x1, x, w3p, a3s, a3d, d1, d2, n_h2, n_h3):
  n = N_NODES
  d3 = w3p.shape[0]
  p3_tot = w3p.shape[1]

  def body(acc_ref, s_ref, b2_ref, x1_ref, x_ref, w3_ref, as_ref, ad_ref,
           *out_refs):
    s = s_ref[:, 0]
    parts = [(acc_ref[2 * hh] + acc_ref[2 * hh + 1])[:, :_hw(d2, hh)]
             for hh in range(n_h2)]
    agg = jnp.concatenate(parts, axis=1) if n_h2 > 1 else parts[0]
    x2 = jax.nn.relu(agg / s[:, None] + b2_ref[...])
    cat = jnp.concatenate([x1_ref[...], x_ref[...], x2], axis=1)
    h3 = jnp.dot(cat, w3_ref[...], preferred_element_type=_f32)
    for hh in range(n_h3):
      out_refs[hh][...] = h3[:, hh * 128:(hh + 1) * 128]
    asrc = jnp.dot(h3, as_ref[...], preferred_element_type=_f32)
    adst = jnp.dot(h3, ad_ref[...], preferred_element_type=_f32)
    out_refs[n_h3][...] = jnp.broadcast_to(asrc, (_RB, 8))
    out_refs[n_h3 + 1][...] = jnp.broadcast_to(adst, (_RB, 8))

  out_specs = ([pl.BlockSpec((_RB, 128), lambda i: (i, 0))] * n_h3 +
               [pl.BlockSpec((_RB, 8), lambda i: (i, 0))] * 2)
  out_shape = ([jax.ShapeDtypeStruct((n, 128), _f32)] * n_h3 +
               [jax.ShapeDtypeStruct((n, 8), _f32)] * 2)
  return pl.pallas_call(
      body,
      grid=(n // _RB,),
      in_specs=[pl.BlockSpec((2 * n_h2, _RB, 128), lambda i: (0, i, 0)),
                pl.BlockSpec((_RB, 8), lambda i: (i, 0)),
                pl.BlockSpec((1, d2), lambda i: (0, 0)),
                pl.BlockSpec((_RB, d1), lambda i: (i, 0)),
                pl.BlockSpec((_RB, d1), lambda i: (i, 0)),
                pl.BlockSpec((d3, p3_tot), lambda i: (0, 0)),
                pl.BlockSpec((p3_tot, 1), lambda i: (0, 0)),
                pl.BlockSpec((p3_tot, 1), lambda i: (0, 0))],
      out_specs=out_specs,
      out_shape=out_shape,
  )(acc, s_part, b2, x1, x, w3p, a3s, a3d)


def _tc_gat3_pool(acc, s_part, b3, batch3d, d3, n_h3):
  n = N_NODES
  B = 128

  def body(acc_ref, s_ref, b3_ref, bt_ref, o_ref):
    i = pl.program_id(0)
    s = s_ref[:, 0]
    parts = []
    for hh in range(n_h3):
      agg = acc_ref[2 * hh] + acc_ref[2 * hh + 1]
      parts.append(agg[:, :_hw(d3, hh)])
    agg = jnp.concatenate(parts, axis=1)
    x3 = jax.nn.relu(agg / s[:, None] + b3_ref[...])
    bt = bt_ref[0, 0, :]
    oh = (bt[:, None] == lax.broadcasted_iota(jnp.int32, (_RB, B), 1))
    oh = oh.astype(_f32)
    x3c = jnp.concatenate([x3, jnp.ones((_RB, 1), _f32)], axis=1)
    part = lax.dot_general(oh, x3c, (((0,), (0,)), ((), ())),
                           precision=lax.Precision.HIGHEST,
                           preferred_element_type=_f32)

    @pl.when(i == 0)
    def _():
      o_ref[...] = jnp.zeros_like(o_ref)
    o_ref[...] += part

  return pl.pallas_call(
      body,
      grid=(n // _RB,),
      in_specs=[pl.BlockSpec((2 * n_h3, _RB, 128), lambda i: (0, i, 0)),
                pl.BlockSpec((_RB, 8), lambda i: (i, 0)),
                pl.BlockSpec((1, d3), lambda i: (0, 0)),
                pl.BlockSpec((1, 1, _RB), lambda i: (i, 0, 0))],
      out_specs=pl.BlockSpec((128, d3 + 1), lambda i: (0, 0)),
      out_shape=jax.ShapeDtypeStruct((128, d3 + 1), _f32),
  )(acc, s_part, b3, batch3d)


def _ln(x, g, b):
  m = jnp.mean(x, axis=-1, keepdims=True)
  v = jnp.mean((x - m) ** 2, axis=-1, keepdims=True)
  return g * (x - m) / jnp.sqrt(v + 1e-5) + b


def _bn(x, g, b):
  m = jnp.mean(x, axis=0)
  v = jnp.mean((x - m) ** 2, axis=0)
  return g * (x - m) / jnp.sqrt(v + 1e-5) + b


def _tc_fusion(pooled_mol, pooled_pro, smiles, sequence, e3fp, ergfp, pubfp,
               maccsfp, smi_m, seq_m, pf):
  B = 128
  names = ['mol_graph', 'pro_graph', 'smiles_seq', 'protein_seq', 'smi_m',
           'seq_m', 'e3fp', 'ergfp', 'pubfp', 'maccsfp']

  p = pf
  flat = [
      p['mol_fc1']['w'], p['mol_fc1']['b'][None, :],
      p['mol_fc2']['w'], p['mol_fc2']['b'][None, :],
      p['pro_fc1']['w'], p['pro_fc1']['b'][None, :],
      p['pro_fc2']['w'], p['pro_fc2']['b'][None, :],
      p['sm_emb'], p['sm_fc']['w'], p['sm_fc']['b'][None, :],
      p['se_emb'], p['se_fc']['w'], p['se_fc']['b'][None, :],
      p['q_fc']['w'], p['q_fc']['b'][None, :],
      p['d_fc']['w'], p['d_fc']['b'][None, :],
      p['f1']['w'], p['f1']['b'][None, :],
      p['f2']['w'], p['f2']['b'][None, :],
      p['f3']['w'], p['f3']['b'][None, :],
      p['f4']['w'], p['f4']['b'][None, :],
      jnp.stack([p['ft'][n]['w'] for n in names]),
      jnp.stack([p['ft'][n]['b'] for n in names]),
      jnp.stack([p['ft'][n]['g'] for n in names]),
      jnp.stack([p['ft'][n]['be'] for n in names]),
      jnp.stack([p['gate'][n]['w'][:, 0] for n in names]),
      jnp.stack([p['gate'][n]['b'] for n in names]),
      p['op1']['w'], p['op1']['b'][None, :],
      p['op2']['w'], p['op2']['b'][None, :],
      p['fc1']['w'], p['fc1']['b'][None, :],
      p['bn1']['g'][None, :], p['bn1']['be'][None, :],
      p['fc2']['w'], p['fc2']['b'][None, :],
      p['bn2']['g'][None, :], p['bn2']['be'][None, :],
      p['out']['w'], p['out']['b'][None, :],
      p['bno']['g'][None, :], p['bno']['be'][None, :],
  ]

  def body(pm_ref, pp_ref, sm_ref, se_ref, e3_ref, er_ref, pu_ref, ma_ref,
           qm_ref, dm_ref,
           mf1w, mf1b, mf2w, mf2b, pf1w, pf1b, pf2w, pf2b,
           smemb, smfw, smfb, seemb, sefw, sefb,
           qw, qb, dw, db, f1w, f1b, f2w, f2b, f3w, f3b, f4w, f4b,
           ftw, ftb, ftg, ftbe, gww, gwb,
           op1w, op1b, op2w, op2b, fc1w, fc1b, bn1g, bn1b,
           fc2w, fc2b, bn2g, bn2b, outw, outb, bnog, bnob,
           o_ref):
    r = jax.nn.relu

    def lin(x, w, b):
      return jnp.dot(x, w[...], preferred_element_type=_f32) + b[...]

    pm = pm_ref[...]
    dm_mol = pm.shape[1] - 1
    mg_mean = pm[:, :dm_mol] / jnp.maximum(pm[:, dm_mol:], 1.0)
    mg = lin(r(lin(mg_mean, mf1w, mf1b)), mf2w, mf2b)

    pp = pp_ref[...]
    dm_pro = pp.shape[1] - 1
    pg_mean = pp[:, :dm_pro] / jnp.maximum(pp[:, dm_pro:], 1.0)
    pg = lin(r(lin(pg_mean, pf1w, pf1b)), pf2w, pf2b)

    def hist(tok_ref, vocab):
      cnt = jnp.zeros((B, vocab), _f32)
      for c in range(8):
        tok = tok_ref[:, c * 128:(c + 1) * 128]
        oh = (tok[:, :, None] ==
              lax.broadcasted_iota(jnp.int32, (B, 128, vocab), 2))
        cnt = cnt + jnp.sum(oh.astype(_f32), axis=1)
      return cnt

    sm_mean = jnp.dot(hist(sm_ref, 64), smemb[...],
                      precision=lax.Precision.HIGHEST,
                      preferred_element_type=_f32) * (1.0 / 1024.0)
    sm = lin(sm_mean, smfw, smfb)
    se_mean = jnp.dot(hist(se_ref, 26), seemb[...],
                      precision=lax.Precision.HIGHEST,
                      preferred_element_type=_f32) * (1.0 / 1024.0)
    se = lin(se_mean, sefw, sefb)

    feats = [mg, pg, sm, se,
             lin(qm_ref[...], qw, qb), lin(dm_ref[...], dw, db),
             lin(e3_ref[...], f1w, f1b), lin(er_ref[...], f2w, f2b),
             lin(pu_ref[...], f3w, f3b), lin(ma_ref[...], f4w, f4b)]

    trs = []
    gcols = []
    for i in range(10):
      t = r(jnp.dot(feats[i], ftw[i], preferred_element_type=_f32) +
            ftb[i][None, :])
      t = _ln(t, ftg[i][None, :], ftbe[i][None, :])
      trs.append(t)
      gcols.append(jnp.sum(t * gww[i][None, :], axis=1, keepdims=True) +
                   gwb[i][None, :])
    gw = jax.nn.softmax(jax.nn.sigmoid(jnp.concatenate(gcols, axis=1)),
                        axis=1)
    fused = jnp.concatenate(
        [gw[:, i:i + 1] * trs[i] for i in range(10)], axis=1)
    fused = lin(r(lin(fused, op1w, op1b)), op2w, op2b)
    xc = r(_bn(lin(fused, fc1w, fc1b), bn1g[...], bn1b[...]))
    xc = r(_bn(lin(xc, fc2w, fc2b), bn2g[...], bn2b[...]))
    o_ref[...] = _bn(lin(xc, outw, outb), bnog[...], bnob[...])

  ins = [pooled_mol, pooled_pro, smiles, sequence, e3fp, ergfp, pubfp,
         maccsfp, smi_m, seq_m] + flat
  return pl.pallas_call(
      body,
      out_shape=jax.ShapeDtypeStruct((B, 1), _f32),
      compiler_params=pltpu.CompilerParams(
          vmem_limit_bytes=100 * 1024 * 1024),
  )(*ins)


def _prep_edges(ei):
  loop = jnp.arange(N_NODES, dtype=jnp.int32)
  src = jnp.concatenate([ei[0].astype(jnp.int32), loop])
  dst = jnp.concatenate([ei[1].astype(jnp.int32), loop])
  pad = E_PAD - E_REAL
  src = jnp.concatenate([src, jnp.zeros((pad,), jnp.int32)])
  dst = jnp.concatenate([dst, jnp.zeros((pad,), jnp.int32)])
  return (src.reshape(NW, NCH, K_EDGE), dst.reshape(NW, NCH, K_EDGE))


def _pad_w(w, rows, cols):
  return jnp.pad(w, ((0, rows - w.shape[0]), (0, cols - w.shape[1])))


def _gnn_stack(x, ei, batch, pc1, pc2, pc3, d1):
  d2, d3 = 2 * d1, 4 * d1
  n_h2 = (d2 + 127) // 128
  n_h3 = (d3 + 127) // 128

  src, dst = _prep_edges(ei)
  zeros_n = jnp.zeros((N_NODES,), _f32)

  _, deg_part = _sc_edges(src, dst, zeros_n, zeros_n, (),
                          pd=16, n_halves=0, mode="deg")
  dinv = _tc_sum32(deg_part, rsqrt=True)[:, 0]

  h1 = _tc_matmul(x, _pad_w(pc1['w'], d1, 128))
  acc1, _ = _sc_edges(src, dst, dinv, dinv, (h1,),
                      pd=128, n_halves=1, mode="gcn")

  w2p = _pad_w(pc2['w'], d2, 128 * n_h2)
  a2s = _pad_w(pc2['a_src'][:, None], 128 * n_h2, 1)
  a2d = _pad_w(pc2['a_dst'][:, None], 128 * n_h2, 1)
  outs2 = _tc_gcn_finish(acc1, pc1['b'][None, :], x, w2p, a2s, a2d,
                         d1, n_h2)
  x1 = outs2[0]
  h2s = tuple(outs2[1:1 + n_h2])
  asrc2, adst2 = outs2[1 + n_h2], outs2[2 + n_h2]

  e2 = _sc_edges(src, dst, asrc2[:, 0], adst2[:, 0], (),
                 pd=16, n_halves=0, mode="logit")
  acc2, s2 = _sc_edges(src, dst, zeros_n, zeros_n, h2s,
                       pd=128, n_halves=n_h2, mode="gat", wtab=_tc_exp(e2))

  w3p = _pad_w(pc3['w'], d3, 128 * n_h3)
  a3s = _pad_w(pc3['a_src'][:, None], 128 * n_h3, 1)
  a3d = _pad_w(pc3['a_dst'][:, None], 128 * n_h3, 1)
  outs3 = _tc_gat_finish(acc2, _tc_sum32(s2), pc2['b'][None, :], x1, x,
                         w3p, a3s, a3d, d1, d2, n_h2, n_h3)
  h3s = tuple(outs3[:n_h3])
  asrc3, adst3 = outs3[n_h3], outs3[n_h3 + 1]

  e3 = _sc_edges(src, dst, asrc3[:, 0], adst3[:, 0], (),
                 pd=16, n_halves=0, mode="logit")
  acc3, s3 = _sc_edges(src, dst, zeros_n, zeros_n, h3s,
                       pd=128, n_halves=n_h3, mode="gat", wtab=_tc_exp(e3))

  batch3d = batch.astype(jnp.int32).reshape(N_NODES // _RB, 1, _RB)
  pooled = _tc_gat3_pool(acc3, _tc_sum32(s3), pc3['b'][None, :], batch3d,
                         d3, n_h3)
  return pooled


def kernel(smiles, sequence, e3fp, ergfp, pubfp, maccsfp, mol_x,
           mol_edge_index, mol_batch, pro_x, pro_edge_index, pro_batch,
           smi_m, seq_m, params):
  p = params
  pooled_mol = _gnn_stack(mol_x, mol_edge_index, mol_batch,
                          p['mol_c1'], p['mol_c2'], p['mol_c3'], 78)
  pooled_pro = _gnn_stack(pro_x, pro_edge_index, pro_batch,
                          p['pro_c1'], p['pro_c2'], p['pro_c3'], 33)

  pf = {k: p[k] for k in
        ['mol_fc1', 'mol_fc2', 'pro_fc1', 'pro_fc2', 'sm_emb', 'sm_fc',
         'se_emb', 'se_fc', 'q_fc', 'd_fc', 'f1', 'f2', 'f3', 'f4',
         'ft', 'gate', 'op1', 'op2', 'fc1', 'bn1', 'fc2', 'bn2',
         'out', 'bno']}
  return _tc_fusion(pooled_mol, pooled_pro, smiles.astype(jnp.int32),
                    sequence.astype(jnp.int32), e3fp, ergfp, pubfp, maccsfp,
                    smi_m, seq_m, pf)
